# baseline (speedup 1.0000x reference)
import functools

import jax
import jax.numpy as jnp
from jax import lax
from jax.experimental import pallas as pl
from jax.experimental.pallas import tpu as pltpu
from jax.experimental.pallas import tpu_sc as plsc


def _add_body(x_ref, e_ref, o_ref):
    o_ref[...] = x_ref[...] + e_ref[...]


def _kernel_tc(x, emb):
    B, T, H = x.shape
    bt = 512
    bb = 2

    return pl.pallas_call(
        _add_body,
        grid=(T // bt, B // bb),
        in_specs=[
            pl.BlockSpec((bb, bt, H), lambda t, b: (b, t, 0)),
            pl.BlockSpec((bt, H), lambda t, b: (t, 0)),
        ],
        out_specs=pl.BlockSpec((bb, bt, H), lambda t, b: (b, t, 0)),
        out_shape=jax.ShapeDtypeStruct(x.shape, x.dtype),
    )(x, emb[:T])


_NW = 32
_NT = 8


def _compute_add(xb, eb):
    def row_loop(r, c):
        xr = xb.at[r]
        er = eb.at[r]

        def col_loop(j, c2):
            base_c = pl.multiple_of(j * 128, 128)
            for k in range(8):
                sl = pl.ds(base_c + k * 16, 16)
                plsc.addupdate(xr.at[sl], er[sl])
            return c2

        return lax.fori_loop(0, 16, col_loop, c)

    lax.fori_loop(0, _NT, row_loop, 0)


def _sc_body(B, T, H, x_hbm, emb_hbm, out_hbm,
             eb0, eb1, xb0, xb1, xb2, xb3,
             es0, es1, xs0, xs1, xs2, xs3, os0, os1, os2, os3):
    wid = lax.axis_index("s") * 2 + lax.axis_index("c")
    t_per_w = T // _NW
    base = wid * t_per_w
    n_tiles = t_per_w // _NT

    ebufs, esems = [eb0, eb1], [es0, es1]
    xbufs, xsems = [xb0, xb1, xb2, xb3], [xs0, xs1, xs2, xs3]
    osems = [os0, os1, os2, os3]
    units = [(t, b) for t in range(n_tiles) for b in range(B)]

    def x_in(u):
        t, b = units[u]
        return pltpu.async_copy(
            x_hbm.at[b, pl.ds(base + t * _NT, _NT)], xbufs[u % 4], xsems[u % 4])

    def e_in(t):
        return pltpu.async_copy(
            emb_hbm.at[pl.ds(base + t * _NT, _NT)], ebufs[t % 2], esems[t % 2])

    e_descs = {0: e_in(0), 1: e_in(1)}
    x_descs = {0: x_in(0), 1: x_in(1)}
    o_descs = {}
    for u, (t, b) in enumerate(units):
        if b == 0:
            e_descs[t].wait()
        x_descs[u].wait()
        _compute_add(xbufs[u % 4], ebufs[t % 2])
        o_descs[u] = pltpu.async_copy(
            xbufs[u % 4], out_hbm.at[b, pl.ds(base + t * _NT, _NT)], osems[u % 4])
        if u + 2 < len(units):
            if u - 2 >= 0:
                o_descs[u - 2].wait()
            x_descs[u + 2] = x_in(u + 2)
        if b == B - 1 and t + 2 < n_tiles:
            e_descs[t + 2] = e_in(t + 2)
    o_descs[len(units) - 2].wait()
    o_descs[len(units) - 1].wait()


def _kernel_sc(x, emb):
    B, T, H = x.shape
    mesh = plsc.VectorSubcoreMesh(core_axis_name="c", subcore_axis_name="s")
    k = functools.partial(
        pl.kernel,
        mesh=mesh,
        out_type=jax.ShapeDtypeStruct((B, T, H), x.dtype),
        scratch_types=(
            [pltpu.VMEM((_NT, H), jnp.float32)] * 6
            + [pltpu.SemaphoreType.DMA] * 10
        ),
    )(functools.partial(_sc_body, B, T, H))
    return k(x, emb[:T])


def kernel(x, emb):
    return _kernel_sc(x, emb)


# speedup vs baseline: 3.2607x; 3.2607x over previous
import jax
import jax.numpy as jnp
from jax.experimental import pallas as pl


def _add_body(x_ref, e_ref, o_ref):
    o_ref[...] = x_ref[...] + e_ref[...]


def kernel(x, emb):
    B, T, H = x.shape
    bt = 512
    bb = 2

    return pl.pallas_call(
        _add_body,
        grid=(T // bt, B // bb),
        in_specs=[
            pl.BlockSpec((bb, bt, H), lambda t, b: (b, t, 0)),
            pl.BlockSpec((bt, H), lambda t, b: (t, 0)),
        ],
        out_specs=pl.BlockSpec((bb, bt, H), lambda t, b: (b, t, 0)),
        out_shape=jax.ShapeDtypeStruct(x.shape, x.dtype),
    )(x, emb[:T])
